# Initial kernel scaffold; baseline (speedup 1.0000x reference)
import jax
import jax.numpy as jnp
from jax.experimental import pallas as pl


def kernel(x, table, pe):
    raise NotImplementedError("write your pallas kernel here")


# speedup vs baseline: 1.9689x; 1.9689x over previous
import jax
import jax.numpy as jnp
from jax import lax
from jax.experimental import pallas as pl
from jax.experimental.pallas import tpu as pltpu
from jax.experimental.pallas import tpu_sc as plsc

B, L, D, V = 4096, 200, 128, 100000
NC, NS, LANES = 2, 16, 16
NW = NC * NS
BPW = B // NW
NCHUNK = D // LANES


def _emb_body(xt_hbm, table_hbm, pe_hbm, out_hbm, idx_v, pe_v, buf_v, sem_g):
    wid = lax.axis_index("s") * NC + lax.axis_index("c")
    b0 = wid * BPW

    pltpu.sync_copy(xt_hbm.at[:, pl.ds(b0, BPW)], idx_v)
    pltpu.sync_copy(pe_hbm, pe_v)

    def body(l, carry):
        pltpu.async_copy(table_hbm.at[idx_v.at[l]], buf_v, sem_g).wait()

        for c in range(NCHUNK):
            pev = pe_v[l, pl.ds(c * LANES, LANES)]

            def inner(b, cc, pev=pev, c=c):
                sl = pl.ds(c * LANES, LANES)
                buf_v[b, sl] = buf_v[b, sl] + pev
                return cc

            lax.fori_loop(0, BPW, inner, 0)

        pltpu.sync_copy(buf_v, out_hbm.at[pl.ds(b0, BPW), l])
        return carry

    lax.fori_loop(0, L, body, 0)


def kernel(x, table, pe):
    xt = x.T
    pe2 = pe.reshape(L, D)
    run = pl.kernel(
        _emb_body,
        out_type=jax.ShapeDtypeStruct((B, L, D), jnp.float32),
        mesh=plsc.VectorSubcoreMesh(core_axis_name="c", subcore_axis_name="s"),
        scratch_types=[
            pltpu.VMEM((L, BPW), jnp.int32),
            pltpu.VMEM((L, D), jnp.float32),
            pltpu.VMEM((BPW, D), jnp.float32),
            pltpu.SemaphoreType.DMA,
        ],
    )
    return run(xt, table, pe2)


# speedup vs baseline: 2.7641x; 1.4039x over previous
import jax
import jax.numpy as jnp
from jax import lax
from jax.experimental import pallas as pl
from jax.experimental.pallas import tpu as pltpu
from jax.experimental.pallas import tpu_sc as plsc

B, L, D, V = 4096, 200, 128, 100000
NC, NS, LANES = 2, 16, 16
NW = NC * NS
BPW = B // NW
NCHUNK = D // LANES
NBUF = 4


def _emb_body(xt_hbm, table_hbm, pe_hbm, out_hbm, idx_v, pe_v, bufs, semg, sems):
    wid = lax.axis_index("s") * NC + lax.axis_index("c")
    b0 = wid * BPW

    pltpu.sync_copy(xt_hbm.at[:, pl.ds(b0, BPW)], idx_v)
    pltpu.sync_copy(pe_hbm, pe_v)

    def gather_fire(l, j):
        pltpu.async_copy(table_hbm.at[idx_v.at[l]], bufs[j], semg[j])

    def gather_wait(l, j):
        pltpu.make_async_copy(table_hbm.at[idx_v.at[l]], bufs[j], semg[j]).wait()

    def store_fire(l, j):
        pltpu.async_copy(bufs[j], out_hbm.at[pl.ds(b0, BPW), l], sems[j])

    def store_wait(l, j):
        pltpu.make_async_copy(bufs[j], out_hbm.at[pl.ds(b0, BPW), l], sems[j]).wait()

    def add_pe(l, j):
        buf = bufs[j]
        for c in range(NCHUNK):
            pev = pe_v[l, pl.ds(c * LANES, LANES)]

            def inner(b, cc, pev=pev, c=c):
                sl = pl.ds(c * LANES, LANES)
                buf[b, sl] = buf[b, sl] + pev
                return cc

            lax.fori_loop(0, BPW, inner, 0)

    for j in range(NBUF - 1):
        gather_fire(j, j)

    def outer(i, carry):
        base = i * NBUF
        for jj in range(NBUF):
            l = base + jj
            gather_wait(l, jj)
            add_pe(l, jj)
            store_fire(l, jj)
            jp = (jj - 1) % NBUF
            store_wait(l - 1, jp)
            gather_fire(l + NBUF - 1, jp)
        return carry

    for jj in range(NBUF):
        l = jj
        gather_wait(l, jj)
        add_pe(l, jj)
        store_fire(l, jj)
        jp = (jj - 1) % NBUF
        if jj > 0:
            store_wait(l - 1, jp)
        gather_fire(l + NBUF - 1, jp)

    lax.fori_loop(1, L // NBUF - 1, outer, 0)

    base = L - NBUF
    for jj in range(NBUF):
        l = base + jj
        gather_wait(l, jj)
        add_pe(l, jj)
        store_fire(l, jj)
        jp = (jj - 1) % NBUF
        store_wait(l - 1, jp)
        if l + NBUF - 1 < L:
            gather_fire(l + NBUF - 1, jp)
    store_wait(L - 1, (NBUF - 1) % NBUF)


def kernel(x, table, pe):
    xt = x.T
    pe2 = pe.reshape(L, D)
    run = pl.kernel(
        _emb_body,
        out_type=jax.ShapeDtypeStruct((B, L, D), jnp.float32),
        mesh=plsc.VectorSubcoreMesh(core_axis_name="c", subcore_axis_name="s"),
        scratch_types=[
            pltpu.VMEM((L, BPW), jnp.int32),
            pltpu.VMEM((L, D), jnp.float32),
            [pltpu.VMEM((BPW, D), jnp.float32) for _ in range(NBUF)],
            [pltpu.SemaphoreType.DMA for _ in range(NBUF)],
            [pltpu.SemaphoreType.DMA for _ in range(NBUF)],
        ],
    )
    return run(xt, table, pe2)


# speedup vs baseline: 9.4665x; 3.4248x over previous
import jax
import jax.numpy as jnp
from jax import lax
from jax.experimental import pallas as pl
from jax.experimental.pallas import tpu as pltpu
from jax.experimental.pallas import tpu_sc as plsc

B, L, D, V = 4096, 200, 128, 100000
NC, NS, LANES = 2, 16, 16
NW = NC * NS
BPW = B // NW
NCHUNK = D // LANES
NBUF = 4


def _emb_body(xt_hbm, table_hbm, pe_hbm, out_hbm, idx_v, pe_v, bufs, semg, sems):
    wid = lax.axis_index("s") * NC + lax.axis_index("c")
    b0 = wid * BPW

    pltpu.sync_copy(xt_hbm.at[:, pl.ds(b0, BPW)], idx_v)
    pltpu.sync_copy(pe_hbm, pe_v)

    def gather_fire(l, j):
        pltpu.async_copy(table_hbm.at[idx_v.at[l]], bufs[j], semg[j])

    def gather_wait(l, j):
        pltpu.make_async_copy(table_hbm.at[idx_v.at[l]], bufs[j], semg[j]).wait()

    def store_fire(l, j):
        pltpu.async_copy(bufs[j], out_hbm.at[pl.ds(b0, BPW), l], sems[j])

    def store_wait(l, j):
        pltpu.make_async_copy(bufs[j], out_hbm.at[pl.ds(b0, BPW), l], sems[j]).wait()

    def add_pe(l, j):
        buf = bufs[j]
        pevs = [pe_v[l, pl.ds(c * LANES, LANES)] for c in range(NCHUNK)]

        @plsc.parallel_loop(0, BPW, 1, unroll=4)
        def _body(b):
            for c in range(NCHUNK):
                sl = pl.ds(c * LANES, LANES)
                buf[b, sl] = buf[b, sl] + pevs[c]

    for j in range(NBUF - 1):
        gather_fire(j, j)

    def outer(i, carry):
        base = i * NBUF
        for jj in range(NBUF):
            l = base + jj
            gather_wait(l, jj)
            add_pe(l, jj)
            store_fire(l, jj)
            jp = (jj - 1) % NBUF
            store_wait(l - 1, jp)
            gather_fire(l + NBUF - 1, jp)
        return carry

    for jj in range(NBUF):
        l = jj
        gather_wait(l, jj)
        add_pe(l, jj)
        store_fire(l, jj)
        jp = (jj - 1) % NBUF
        if jj > 0:
            store_wait(l - 1, jp)
        gather_fire(l + NBUF - 1, jp)

    lax.fori_loop(1, L // NBUF - 1, outer, 0)

    base = L - NBUF
    for jj in range(NBUF):
        l = base + jj
        gather_wait(l, jj)
        add_pe(l, jj)
        store_fire(l, jj)
        jp = (jj - 1) % NBUF
        store_wait(l - 1, jp)
        if l + NBUF - 1 < L:
            gather_fire(l + NBUF - 1, jp)
    store_wait(L - 1, (NBUF - 1) % NBUF)


def kernel(x, table, pe):
    xt = x.T
    pe2 = pe.reshape(L, D)
    run = pl.kernel(
        _emb_body,
        out_type=jax.ShapeDtypeStruct((B, L, D), jnp.float32),
        mesh=plsc.VectorSubcoreMesh(core_axis_name="c", subcore_axis_name="s"),
        scratch_types=[
            pltpu.VMEM((L, BPW), jnp.int32),
            pltpu.VMEM((L, D), jnp.float32),
            [pltpu.VMEM((BPW, D), jnp.float32) for _ in range(NBUF)],
            [pltpu.SemaphoreType.DMA for _ in range(NBUF)],
            [pltpu.SemaphoreType.DMA for _ in range(NBUF)],
        ],
    )
    return run(xt, table, pe2)
